# baseline (speedup 1.0000x reference)
import jax
import jax.numpy as jnp
from jax.experimental import pallas as pallas
from jax.experimental.pallas import tpu as pltpu

APn = 256
UEn = 2048
HID = 128
NS = 10

F32 = jnp.float32
BF16 = jnp.bfloat16


def _mm(a, b):
    return jax.lax.dot_general(a, b, (((1,), (0,)), ((), ())),
                               preferred_element_type=F32)


def _mm_nt(a, b):
    return jax.lax.dot_general(a, b, (((1,), (1,)), ((), ())),
                               preferred_element_type=F32)


def _build_maskT(adjT, n_rows, n_cols):
    iota = jax.lax.broadcasted_iota(jnp.int16, (n_rows, n_cols), 0)
    acc = adjT[0:1, :] == iota
    for k in range(1, NS):
        acc = jnp.logical_or(acc, adjT[k:k + 1, :] == iota)
    return acc


def _fused_kernel(p_hbm, adjT_ue_ref, adjT_ap_ref,
                  w1_hbm, w2_hbm, w3_hbm, w4_hbm, w5_hbm, out_ref,
                  p_v, w1_v, w2_v, w3_v, w4_v, w5_v, sems):
    copies = []
    for i, (src, dst) in enumerate([
            (p_hbm, p_v), (w1_hbm, w1_v), (w2_hbm, w2_v),
            (w3_hbm, w3_v), (w4_hbm, w4_v), (w5_hbm, w5_v)]):
        cp = pltpu.make_async_copy(src, dst, sems.at[i])
        cp.start()
        copies.append(cp)

    m1T = _build_maskT(adjT_ap_ref[...], UEn, APn).astype(BF16)
    m3T = _build_maskT(adjT_ue_ref[...], APn, UEn).astype(BF16)
    d1 = jnp.sum(m1T, axis=0).astype(F32)
    d3 = jnp.sum(m3T, axis=0).astype(F32)
    inv1 = (1.0 / (d1 + 1.0))[None, :]
    inv3 = (1.0 / (d3 + 1.0))[None, :]

    m2T = jnp.minimum(_mm(m3T, m1T), 1.0).astype(BF16)
    d2 = jnp.sum(m2T, axis=0).astype(F32)
    inv2 = (1.0 / (d2 + 1.0))[None, :]
    m4T = jnp.minimum(_mm(m1T, m3T).astype(BF16), BF16(1.0))

    for cp in copies:
        cp.wait()

    Pb = p_v[...].astype(BF16)
    W1 = w1_v[...].astype(BF16)
    W2 = w2_v[...].astype(BF16)
    W3 = w3_v[...].astype(BF16)
    W4 = w4_v[...].astype(BF16)
    W5 = w5_v[...].astype(BF16)

    s1 = _mm_nt(W1[:, :UEn], Pb)
    s2 = _mm_nt(W2[:, :UEn], Pb)
    s3 = _mm(W3[:, :APn], Pb)
    s4 = _mm(W4[:, :APn], Pb)
    t1 = _mm(W1[:, UEn:], Pb).astype(BF16)
    t2 = _mm_nt(W2[:, UEn:], Pb).astype(BF16)
    t3 = _mm_nt(W3[:, APn:], Pb).astype(BF16)
    u4 = _mm(W4[:, APn:], Pb).astype(BF16)

    x1 = jnp.maximum(s1 + _mm(t1, m1T) * inv1, 0.0)
    x2 = jnp.maximum(s2 + _mm(t2, m2T) * inv2, 0.0)
    x3 = jnp.maximum(s3 + _mm(t3, m3T) * inv3, 0.0)
    u4e = jnp.concatenate([u4, jnp.ones((8, UEn), BF16)], axis=0)
    z4e = _mm(u4e, m4T)
    inv4 = 1.0 / (z4e[HID:HID + 1] + 1.0)
    x4 = jnp.maximum(s4 + z4e[:HID] * inv4, 0.0)

    cat12 = jnp.concatenate([x1, x2], axis=0).astype(BF16)
    cat34 = jnp.concatenate([x3, x4], axis=0).astype(BF16)
    n5 = _mm(cat34, m1T).astype(BF16)
    x5 = jnp.maximum(_mm(W5[:, :2 * HID], cat12) +
                     _mm(W5[:, 2 * HID:], n5) * inv1, 0.0)
    out_ref[...] = x5


def kernel(pl, require, adj_ue, adj_ap, W1, W2, W3, W4, W5):
    del require
    hbm = pallas.BlockSpec(memory_space=pltpu.MemorySpace.HBM)
    vmem = pallas.BlockSpec(memory_space=pltpu.MemorySpace.VMEM)
    return pallas.pallas_call(
        _fused_kernel,
        out_shape=jax.ShapeDtypeStruct((HID, APn), F32),
        in_specs=[hbm, vmem, vmem, hbm, hbm, hbm, hbm, hbm],
        scratch_shapes=[
            pltpu.VMEM((APn, UEn), F32),
            pltpu.VMEM((HID, UEn + APn), F32),
            pltpu.VMEM((HID, 2 * UEn), F32),
            pltpu.VMEM((HID, APn + UEn), F32),
            pltpu.VMEM((HID, 2 * APn), F32),
            pltpu.VMEM((HID, 4 * HID), F32),
            pltpu.SemaphoreType.DMA((6,)),
        ],
    )(pl, adj_ue.T.astype(jnp.int16), adj_ap.T.astype(jnp.int16),
      W1, W2, W3, W4, W5)


# speedup vs baseline: 1.2568x; 1.2568x over previous
import jax
import jax.numpy as jnp
from jax.experimental import pallas as pallas
from jax.experimental.pallas import tpu as pltpu

APn = 256
UEn = 2048
HID = 128
NS = 10

F32 = jnp.float32
BF16 = jnp.bfloat16


def _mm(a, b):
    return jax.lax.dot_general(a, b, (((1,), (0,)), ((), ())),
                               preferred_element_type=F32)


def _mm_nt(a, b):
    return jax.lax.dot_general(a, b, (((1,), (1,)), ((), ())),
                               preferred_element_type=F32)


def _build_maskT(adjT, n_rows, n_cols):
    iota = jax.lax.broadcasted_iota(jnp.int16, (n_rows, n_cols), 0)
    acc = adjT[0:1, :] == iota
    for k in range(1, NS):
        acc = jnp.logical_or(acc, adjT[k:k + 1, :] == iota)
    return acc


def _fused_kernel(p_hbm, adjT_ue_ref, adjT_ap_ref,
                  w1_hbm, w2_hbm, w3_hbm, w4_hbm, w5_hbm, out_ref,
                  p_v, w1_v, w2_v, w3_v, w4_v, w5_v, sems):
    copies = []
    for i, (src, dst) in enumerate([
            (p_hbm, p_v), (w1_hbm, w1_v), (w2_hbm, w2_v),
            (w3_hbm, w3_v), (w4_hbm, w4_v), (w5_hbm, w5_v)]):
        cp = pltpu.make_async_copy(src, dst, sems.at[i])
        cp.start()
        copies.append(cp)

    adjT_ap = adjT_ap_ref[...].astype(jnp.int16)
    adjT_ue = adjT_ue_ref[...].astype(jnp.int16)
    m1T = _build_maskT(adjT_ap, UEn, APn).astype(BF16)
    m3T = _build_maskT(adjT_ue, APn, UEn).astype(BF16)
    d1 = jnp.sum(m1T, axis=0).astype(F32)
    d3 = jnp.sum(m3T, axis=0).astype(F32)
    inv1 = (1.0 / (d1 + 1.0))[None, :]
    inv3 = (1.0 / (d3 + 1.0))[None, :]

    m2T = jnp.minimum(_mm(m3T, m1T), 1.0).astype(BF16)
    d2 = jnp.sum(m2T, axis=0).astype(F32)
    inv2 = (1.0 / (d2 + 1.0))[None, :]
    m4T = jnp.minimum(_mm(m1T, m3T).astype(BF16), BF16(1.0))

    for cp in copies:
        cp.wait()

    Pb = p_v[...].astype(BF16)
    W1 = w1_v[...].astype(BF16)
    W2 = w2_v[...].astype(BF16)
    W3 = w3_v[...].astype(BF16)
    W4 = w4_v[...].astype(BF16)
    W5 = w5_v[...].astype(BF16)

    s1 = _mm_nt(W1[:, :UEn], Pb)
    s2 = _mm_nt(W2[:, :UEn], Pb)
    s3 = _mm(W3[:, :APn], Pb)
    s4 = _mm(W4[:, :APn], Pb)
    t1 = _mm(W1[:, UEn:], Pb).astype(BF16)
    t2 = _mm_nt(W2[:, UEn:], Pb).astype(BF16)
    t3 = _mm_nt(W3[:, APn:], Pb).astype(BF16)
    u4 = _mm(W4[:, APn:], Pb).astype(BF16)

    x1 = jnp.maximum(s1 + _mm(t1, m1T) * inv1, 0.0)
    x2 = jnp.maximum(s2 + _mm(t2, m2T) * inv2, 0.0)
    x3 = jnp.maximum(s3 + _mm(t3, m3T) * inv3, 0.0)
    u4e = jnp.concatenate([u4, jnp.ones((8, UEn), BF16)], axis=0)
    z4e = _mm(u4e, m4T)
    inv4 = 1.0 / (z4e[HID:HID + 1] + 1.0)
    x4 = jnp.maximum(s4 + z4e[:HID] * inv4, 0.0)

    cat12 = jnp.concatenate([x1, x2], axis=0).astype(BF16)
    cat34 = jnp.concatenate([x3, x4], axis=0).astype(BF16)
    n5 = _mm(cat34, m1T).astype(BF16)
    x5 = jnp.maximum(_mm(W5[:, :2 * HID], cat12) +
                     _mm(W5[:, 2 * HID:], n5) * inv1, 0.0)
    out_ref[...] = x5


def kernel(pl, require, adj_ue, adj_ap, W1, W2, W3, W4, W5):
    del require
    hbm = pallas.BlockSpec(memory_space=pltpu.MemorySpace.HBM)
    vmem = pallas.BlockSpec(memory_space=pltpu.MemorySpace.VMEM)
    return pallas.pallas_call(
        _fused_kernel,
        out_shape=jax.ShapeDtypeStruct((HID, APn), F32),
        in_specs=[hbm, vmem, vmem, hbm, hbm, hbm, hbm, hbm],
        scratch_shapes=[
            pltpu.VMEM((APn, UEn), F32),
            pltpu.VMEM((HID, UEn + APn), F32),
            pltpu.VMEM((HID, 2 * UEn), F32),
            pltpu.VMEM((HID, APn + UEn), F32),
            pltpu.VMEM((HID, 2 * APn), F32),
            pltpu.VMEM((HID, 4 * HID), F32),
            pltpu.SemaphoreType.DMA((6,)),
        ],
    )(pl, adj_ue.T, adj_ap.T, W1, W2, W3, W4, W5)
